# baseline (device time: 37981 ns/iter reference)
import jax
import jax.numpy as jnp
from jax import lax
from jax.experimental import pallas as pl
from jax.experimental.pallas import tpu as pltpu

B, H, D = 8, 8, 64
KLOC = 512
SCALE = D ** -0.5


def kernel(Q, K, V):
    def body(q_ref, k_ref, v_ref, o_ref,
             acc, stats, peer_acc, peer_stats, send_sems, recv_sems):
        my_x = lax.axis_index("x")
        my_y = lax.axis_index("y")
        my_z = lax.axis_index("z")
        partner = (1 - my_x, my_y, my_z)

        ms, ls, os_ = [], [], []
        for b in range(B):
            qb = q_ref[b, 0]
            kb = k_ref[b]
            s = jnp.sum(qb[None, :, :] * kb, axis=-1) * SCALE
            m = jnp.max(s, axis=0, keepdims=True)
            p = jnp.exp(s - m)
            l = jnp.sum(p, axis=0, keepdims=True)
            ob = jnp.sum(p[:, :, None] * v_ref[b], axis=0)
            ms.append(m)
            ls.append(l)
            os_.append(ob)
        acc[...] = jnp.stack(os_, axis=0)
        stats[0] = jnp.concatenate(ms, axis=0)
        stats[1] = jnp.concatenate(ls, axis=0)

        barrier_sem = pltpu.get_barrier_semaphore()
        pl.semaphore_signal(barrier_sem, inc=1, device_id=partner,
                            device_id_type=pl.DeviceIdType.MESH)
        pl.semaphore_wait(barrier_sem, 1)

        rdma_o = pltpu.make_async_remote_copy(
            src_ref=acc, dst_ref=peer_acc,
            send_sem=send_sems.at[0], recv_sem=recv_sems.at[0],
            device_id=partner, device_id_type=pl.DeviceIdType.MESH)
        rdma_s = pltpu.make_async_remote_copy(
            src_ref=stats, dst_ref=peer_stats,
            send_sem=send_sems.at[1], recv_sem=recv_sems.at[1],
            device_id=partner, device_id_type=pl.DeviceIdType.MESH)
        rdma_o.start()
        rdma_s.start()
        rdma_o.wait()
        rdma_s.wait()

        m_s, l_s = stats[0], stats[1]
        m_p, l_p = peer_stats[0], peer_stats[1]
        m_n = jnp.maximum(m_s, m_p)
        a_s = jnp.exp(m_s - m_n)
        a_p = jnp.exp(m_p - m_n)
        l_n = a_s * l_s + a_p * l_p
        o = (a_s[:, :, None] * acc[...] + a_p[:, :, None] * peer_acc[...]) \
            / l_n[:, :, None]
        o_ref[...] = o[:, None]

    return pl.pallas_call(
        body,
        out_shape=jax.ShapeDtypeStruct((B, 1, H, D), jnp.float32),
        in_specs=[
            pl.BlockSpec(memory_space=pltpu.VMEM),
            pl.BlockSpec(memory_space=pltpu.VMEM),
            pl.BlockSpec(memory_space=pltpu.VMEM),
        ],
        out_specs=pl.BlockSpec(memory_space=pltpu.VMEM),
        scratch_shapes=[
            pltpu.VMEM((B, H, D), jnp.float32),
            pltpu.VMEM((2, B, H), jnp.float32),
            pltpu.VMEM((B, H, D), jnp.float32),
            pltpu.VMEM((2, B, H), jnp.float32),
            pltpu.SemaphoreType.DMA((2,)),
            pltpu.SemaphoreType.DMA((2,)),
        ],
        compiler_params=pltpu.CompilerParams(collective_id=0),
    )(Q, K, V)


# device time: 21483 ns/iter; 1.7680x vs baseline; 1.7680x over previous
import jax
import jax.numpy as jnp
from jax import lax
from jax.experimental import pallas as pl
from jax.experimental.pallas import tpu as pltpu

B, H, D = 8, 8, 64
KLOC = 512
SCALE = D ** -0.5


def kernel(Q, K, V):
    Q2 = Q.reshape(B, H, D)
    K2 = K.reshape(B, KLOC, H * D)
    V2 = V.reshape(B, KLOC, H * D)

    def body(q_ref, k_ref, v_ref, o_ref,
             acc, stats, peer_acc, peer_stats, send_sems, recv_sems):
        my_x = lax.axis_index("x")
        my_y = lax.axis_index("y")
        my_z = lax.axis_index("z")
        partner = (1 - my_x, my_y, my_z)

        colh = lax.broadcasted_iota(jnp.int32, (H, H * D), 1) // D
        rowh = lax.broadcasted_iota(jnp.int32, (H, H * D), 0)
        qmask = (colh == rowh).astype(jnp.float32)
        eye3 = (lax.broadcasted_iota(jnp.int32, (H, H, 1), 0)
                == lax.broadcasted_iota(jnp.int32, (H, H, 1), 1)
                ).astype(jnp.float32)

        ms, ls, os_ = [], [], []
        for b in range(B):
            qb = q_ref[b]
            qblk = jnp.concatenate([qb] * H, axis=1) * qmask
            s = lax.dot_general(
                k_ref[b], qblk, (((1,), (1,)), ((), ())),
                preferred_element_type=jnp.float32) * SCALE
            m = jnp.max(s, axis=0, keepdims=True)
            p = jnp.exp(s - m)
            l = jnp.sum(p, axis=0, keepdims=True)
            t = lax.dot_general(
                p, v_ref[b], (((0,), (0,)), ((), ())),
                preferred_element_type=jnp.float32)
            ob = jnp.sum(t.reshape(H, H, D) * eye3, axis=0)
            ms.append(m)
            ls.append(l)
            os_.append(ob)
        acc[...] = jnp.stack(os_, axis=0)
        stats[0] = jnp.concatenate(ms, axis=0)
        stats[1] = jnp.concatenate(ls, axis=0)

        barrier_sem = pltpu.get_barrier_semaphore()
        pl.semaphore_signal(barrier_sem, inc=1, device_id=partner,
                            device_id_type=pl.DeviceIdType.MESH)
        pl.semaphore_wait(barrier_sem, 1)

        rdma_o = pltpu.make_async_remote_copy(
            src_ref=acc, dst_ref=peer_acc,
            send_sem=send_sems.at[0], recv_sem=recv_sems.at[0],
            device_id=partner, device_id_type=pl.DeviceIdType.MESH)
        rdma_s = pltpu.make_async_remote_copy(
            src_ref=stats, dst_ref=peer_stats,
            send_sem=send_sems.at[1], recv_sem=recv_sems.at[1],
            device_id=partner, device_id_type=pl.DeviceIdType.MESH)
        rdma_o.start()
        rdma_s.start()
        rdma_o.wait()
        rdma_s.wait()

        m_s, l_s = stats[0], stats[1]
        m_p, l_p = peer_stats[0], peer_stats[1]
        m_n = jnp.maximum(m_s, m_p)
        a_s = jnp.exp(m_s - m_n)
        a_p = jnp.exp(m_p - m_n)
        l_n = a_s * l_s + a_p * l_p
        o = (a_s[:, :, None] * acc[...] + a_p[:, :, None] * peer_acc[...]) \
            / l_n[:, :, None]
        o_ref[...] = o[:, None]

    return pl.pallas_call(
        body,
        out_shape=jax.ShapeDtypeStruct((B, 1, H, D), jnp.float32),
        in_specs=[
            pl.BlockSpec(memory_space=pltpu.VMEM),
            pl.BlockSpec(memory_space=pltpu.VMEM),
            pl.BlockSpec(memory_space=pltpu.VMEM),
        ],
        out_specs=pl.BlockSpec(memory_space=pltpu.VMEM),
        scratch_shapes=[
            pltpu.VMEM((B, H, D), jnp.float32),
            pltpu.VMEM((2, B, H), jnp.float32),
            pltpu.VMEM((B, H, D), jnp.float32),
            pltpu.VMEM((2, B, H), jnp.float32),
            pltpu.SemaphoreType.DMA((2,)),
            pltpu.SemaphoreType.DMA((2,)),
        ],
        compiler_params=pltpu.CompilerParams(collective_id=0),
    )(Q2, K2, V2)


# device time: 19643 ns/iter; 1.9336x vs baseline; 1.0937x over previous
import jax
import jax.numpy as jnp
from jax import lax
from jax.experimental import pallas as pl
from jax.experimental.pallas import tpu as pltpu

B, H, D = 8, 8, 64
KLOC = 512
SCALE = D ** -0.5


def kernel(Q, K, V):
    Q2 = Q.reshape(B, H, D)
    K2 = K.reshape(B, KLOC, H * D)
    V2 = V.reshape(B, KLOC, H * D)

    def body(q_ref, k_ref, v_ref, o_ref,
             acc, stats, peer_acc, peer_stats, send_sems, recv_sems):
        my_x = lax.axis_index("x")
        my_y = lax.axis_index("y")
        my_z = lax.axis_index("z")
        partner = (1 - my_x, my_y, my_z)

        colh = lax.broadcasted_iota(jnp.int32, (H, H * D), 1) // D
        rowh = lax.broadcasted_iota(jnp.int32, (H, H * D), 0)
        qmask = (colh == rowh).astype(jnp.float32)
        eye3 = (lax.broadcasted_iota(jnp.int32, (H, H, 1), 0)
                == lax.broadcasted_iota(jnp.int32, (H, H, 1), 1)
                ).astype(jnp.float32)

        ms, ls, os_ = [], [], []
        for b in range(B):
            qb = q_ref[b]
            qblk = jnp.concatenate([qb] * H, axis=1) * qmask
            s = lax.dot_general(
                qblk, k_ref[b], (((1,), (1,)), ((), ())),
                preferred_element_type=jnp.float32) * SCALE
            m = jnp.max(s, axis=1, keepdims=True)
            p = jnp.exp(s - m)
            l = jnp.sum(p, axis=1, keepdims=True)
            t = lax.dot_general(
                p, v_ref[b], (((1,), (0,)), ((), ())),
                preferred_element_type=jnp.float32)
            ob = jnp.sum(t.reshape(H, H, D) * eye3, axis=0)
            ms.append(m.reshape(1, H))
            ls.append(l.reshape(1, H))
            os_.append(ob)
        acc[...] = jnp.stack(os_, axis=0)
        stats[0] = jnp.concatenate(ms, axis=0)
        stats[1] = jnp.concatenate(ls, axis=0)

        barrier_sem = pltpu.get_barrier_semaphore()
        pl.semaphore_signal(barrier_sem, inc=1, device_id=partner,
                            device_id_type=pl.DeviceIdType.MESH)
        pl.semaphore_wait(barrier_sem, 1)

        rdma_o = pltpu.make_async_remote_copy(
            src_ref=acc, dst_ref=peer_acc,
            send_sem=send_sems.at[0], recv_sem=recv_sems.at[0],
            device_id=partner, device_id_type=pl.DeviceIdType.MESH)
        rdma_s = pltpu.make_async_remote_copy(
            src_ref=stats, dst_ref=peer_stats,
            send_sem=send_sems.at[1], recv_sem=recv_sems.at[1],
            device_id=partner, device_id_type=pl.DeviceIdType.MESH)
        rdma_o.start()
        rdma_s.start()
        rdma_o.wait()
        rdma_s.wait()

        m_s, l_s = stats[0], stats[1]
        m_p, l_p = peer_stats[0], peer_stats[1]
        m_n = jnp.maximum(m_s, m_p)
        a_s = jnp.exp(m_s - m_n)
        a_p = jnp.exp(m_p - m_n)
        l_n = a_s * l_s + a_p * l_p
        o = (a_s[:, :, None] * acc[...] + a_p[:, :, None] * peer_acc[...]) \
            / l_n[:, :, None]
        o_ref[...] = o[:, None]

    return pl.pallas_call(
        body,
        out_shape=jax.ShapeDtypeStruct((B, 1, H, D), jnp.float32),
        in_specs=[
            pl.BlockSpec(memory_space=pltpu.VMEM),
            pl.BlockSpec(memory_space=pltpu.VMEM),
            pl.BlockSpec(memory_space=pltpu.VMEM),
        ],
        out_specs=pl.BlockSpec(memory_space=pltpu.VMEM),
        scratch_shapes=[
            pltpu.VMEM((B, H, D), jnp.float32),
            pltpu.VMEM((2, B, H), jnp.float32),
            pltpu.VMEM((B, H, D), jnp.float32),
            pltpu.VMEM((2, B, H), jnp.float32),
            pltpu.SemaphoreType.DMA((2,)),
            pltpu.SemaphoreType.DMA((2,)),
        ],
        compiler_params=pltpu.CompilerParams(collective_id=0),
    )(Q2, K2, V2)


# device time: 14984 ns/iter; 2.5348x vs baseline; 1.3109x over previous
import jax
import jax.numpy as jnp
from jax import lax
from jax.experimental import pallas as pl
from jax.experimental.pallas import tpu as pltpu

B, H, D = 8, 8, 64
KLOC = 512
SCALE = D ** -0.5


def kernel(Q, K, V):
    Q2 = Q.reshape(B, H, D)
    KT = K.reshape(B, KLOC, H * D).transpose(0, 2, 1)
    V2 = V.reshape(B, KLOC, H * D)

    def body(q_ref, k_ref, v_ref, o_ref,
             acc, stats, peer_acc, peer_stats, send_sems, recv_sems):
        my_x = lax.axis_index("x")
        my_y = lax.axis_index("y")
        my_z = lax.axis_index("z")
        partner = (1 - my_x, my_y, my_z)

        colh = lax.broadcasted_iota(jnp.int32, (H, H * D), 1) // D
        rowh = lax.broadcasted_iota(jnp.int32, (H, H * D), 0)
        qmask = (colh == rowh).astype(jnp.float32)
        eye3 = (lax.broadcasted_iota(jnp.int32, (H, H, 1), 0)
                == lax.broadcasted_iota(jnp.int32, (H, H, 1), 1)
                ).astype(jnp.float32)

        ms, ls, os_ = [], [], []
        for b in range(B):
            qb = q_ref[b]
            qblk = jnp.concatenate([qb] * H, axis=1) * qmask
            s = lax.dot_general(
                qblk, k_ref[b], (((1,), (0,)), ((), ())),
                preferred_element_type=jnp.float32) * SCALE
            m = jnp.max(s, axis=1, keepdims=True)
            p = jnp.exp(s - m)
            l = jnp.sum(p, axis=1, keepdims=True)
            t = lax.dot_general(
                p, v_ref[b], (((1,), (0,)), ((), ())),
                preferred_element_type=jnp.float32)
            ob = jnp.sum(t.reshape(H, H, D) * eye3, axis=0)
            ms.append(m.reshape(1, H))
            ls.append(l.reshape(1, H))
            os_.append(ob)
        acc[...] = jnp.stack(os_, axis=0)
        stats[0] = jnp.concatenate(ms, axis=0)
        stats[1] = jnp.concatenate(ls, axis=0)

        barrier_sem = pltpu.get_barrier_semaphore()
        pl.semaphore_signal(barrier_sem, inc=1, device_id=partner,
                            device_id_type=pl.DeviceIdType.MESH)
        pl.semaphore_wait(barrier_sem, 1)

        rdma_o = pltpu.make_async_remote_copy(
            src_ref=acc, dst_ref=peer_acc,
            send_sem=send_sems.at[0], recv_sem=recv_sems.at[0],
            device_id=partner, device_id_type=pl.DeviceIdType.MESH)
        rdma_s = pltpu.make_async_remote_copy(
            src_ref=stats, dst_ref=peer_stats,
            send_sem=send_sems.at[1], recv_sem=recv_sems.at[1],
            device_id=partner, device_id_type=pl.DeviceIdType.MESH)
        rdma_o.start()
        rdma_s.start()
        rdma_o.wait()
        rdma_s.wait()

        m_s, l_s = stats[0], stats[1]
        m_p, l_p = peer_stats[0], peer_stats[1]
        m_n = jnp.maximum(m_s, m_p)
        a_s = jnp.exp(m_s - m_n)
        a_p = jnp.exp(m_p - m_n)
        l_n = a_s * l_s + a_p * l_p
        o = (a_s[:, :, None] * acc[...] + a_p[:, :, None] * peer_acc[...]) \
            / l_n[:, :, None]
        o_ref[...] = o[:, None]

    return pl.pallas_call(
        body,
        out_shape=jax.ShapeDtypeStruct((B, 1, H, D), jnp.float32),
        in_specs=[
            pl.BlockSpec(memory_space=pltpu.VMEM),
            pl.BlockSpec(memory_space=pltpu.VMEM),
            pl.BlockSpec(memory_space=pltpu.VMEM),
        ],
        out_specs=pl.BlockSpec(memory_space=pltpu.VMEM),
        scratch_shapes=[
            pltpu.VMEM((B, H, D), jnp.float32),
            pltpu.VMEM((2, B, H), jnp.float32),
            pltpu.VMEM((B, H, D), jnp.float32),
            pltpu.VMEM((2, B, H), jnp.float32),
            pltpu.SemaphoreType.DMA((2,)),
            pltpu.SemaphoreType.DMA((2,)),
        ],
        compiler_params=pltpu.CompilerParams(collective_id=0),
    )(Q2, KT, V2)
